# Initial kernel scaffold; baseline (speedup 1.0000x reference)
import jax
import jax.numpy as jnp
from jax.experimental import pallas as pl


def kernel(item_idx, emb_table, W, b):
    raise NotImplementedError("write your pallas kernel here")


# speedup vs baseline: 28.5085x; 28.5085x over previous
import functools

import jax
import jax.numpy as jnp
from jax import lax
from jax.experimental import pallas as pl
from jax.experimental.pallas import tpu as pltpu
from jax.experimental.pallas import tpu_sc as plsc

NUM_ITEM = 100000
DIM = 128
BATCH = 4096
HIST = 200

BLK = 2048
NB = 49
N_PAD = NB * BLK

NC, NS, L = 2, 16, 16
NW = NC * NS
B_TOTAL = BATCH * HIST
PER_W = B_TOTAL // NW
CH = 12800
N_CHUNKS = PER_W // CH


def _matvec_body(w_ref, emb_ref, b_ref, out_ref):
    out_ref[...] = (
        lax.dot_general(
            w_ref[...], emb_ref[...],
            dimension_numbers=(((1,), (1,)), ((), ())),
            preferred_element_type=jnp.float32,
        )
        + b_ref[0]
    )[None]


_matvec = pl.pallas_call(
    _matvec_body,
    grid=(NB,),
    in_specs=[
        pl.BlockSpec((1, DIM), lambda g: (0, 0)),
        pl.BlockSpec((BLK, DIM), lambda g: (g, 0)),
        pl.BlockSpec(memory_space=pltpu.SMEM),
    ],
    out_specs=pl.BlockSpec((1, 1, BLK), lambda g: (g, 0, 0)),
    out_shape=jax.ShapeDtypeStruct((NB, 1, BLK), jnp.float32),
)


@functools.partial(
    pl.kernel,
    out_type=jax.ShapeDtypeStruct((B_TOTAL,), jnp.float32),
    mesh=plsc.VectorSubcoreMesh(core_axis_name="c", subcore_axis_name="s"),
    compiler_params=pltpu.CompilerParams(needs_layout_passes=False),
    scratch_types=[
        pltpu.VMEM((N_PAD,), jnp.float32),
        pltpu.VMEM((CH,), jnp.int32),
        pltpu.VMEM((CH,), jnp.float32),
    ],
)
def _gather_scores(scores_hbm, idx_hbm, out_hbm, scores_v, idx_v, out_v):
    wid = lax.axis_index("s") * NC + lax.axis_index("c")
    base = wid * PER_W
    pltpu.sync_copy(scores_hbm, scores_v)

    def gather_16(i, _):
        sl = pl.ds(i * L, L)
        out_v[sl] = plsc.load_gather(scores_v, [idx_v[sl]])
        return 0

    for c in range(N_CHUNKS):
        off = base + c * CH
        pltpu.sync_copy(idx_hbm.at[pl.ds(off, CH)], idx_v)
        lax.fori_loop(0, CH // L, gather_16, 0)
        pltpu.sync_copy(out_v, out_hbm.at[pl.ds(off, CH)])


def kernel(item_idx, emb_table, W, b):
    scores = _matvec(W, emb_table, b)
    scores_flat = scores.reshape(N_PAD)
    idx_flat = item_idx.astype(jnp.int32).reshape(B_TOTAL)
    out = _gather_scores(scores_flat, idx_flat)
    return out.reshape(BATCH, HIST, 1)
